# baseline (speedup 1.0000x reference)
import functools

import jax
import jax.numpy as jnp
from jax import lax
from jax.experimental import pallas as pl
from jax.experimental.pallas import tpu as pltpu
from jax.experimental.pallas import tpu_sc as plsc

_K = 16
_RB = 512
_CH = 16
_BIG = 3.0e38
_NW = 32


def _tc_body(q_ref, pT_ref, ids_ref, pn_ref, nq_ref, br_ref):
    q = q_ref[0]
    pT = pT_ref[0]
    n = pT.shape[1]
    nch = n // _CH
    qx = q[:, 0:1]
    qy = q[:, 1:2]
    qz = q[:, 2:3]
    px = pT[0:1, :]
    py = pT[1:2, :]
    pz = pT[2:3, :]
    nq = (qx * qx + qy * qy) + qz * qz
    np_ = (px * px + py * py) + pz * pz
    pn_ref[0] = np_
    nq_ref[0] = jnp.transpose(nq)
    bq = q.astype(jnp.bfloat16)
    bp = pT.astype(jnp.bfloat16)
    dot = lax.dot_general(bq, bp, (((1,), (0,)), ((), ())),
                          preferred_element_type=jnp.float32)
    x = (nq + np_) - 2.0 * dot
    br_ref[0] = bp.astype(jnp.float32)
    xT = jnp.transpose(x)
    x3 = xT.reshape(nch, _CH, _RB)
    cmT = jnp.min(x3, axis=1)
    iota = lax.broadcasted_iota(jnp.int32, (nch, _RB), 0)
    cols = []
    for _ in range(_K):
        mv = jnp.min(cmT, axis=0, keepdims=True)
        eq = cmT == mv
        idx = jnp.min(jnp.where(eq, iota, jnp.int32(1 << 30)), axis=0,
                      keepdims=True)
        cols.append(idx)
        cmT = jnp.where(iota == idx, _BIG, cmT)
    ids_ref[0] = jnp.concatenate(cols, axis=0)


@jax.jit
def _tc_knn(pts, ptsT):
    P, N, _ = pts.shape
    grid = (P, N // _RB)
    return pl.pallas_call(
        _tc_body,
        grid=grid,
        in_specs=[
            pl.BlockSpec((1, _RB, 3), lambda p, rb: (p, rb, 0)),
            pl.BlockSpec((1, 3, N), lambda p, rb: (p, 0, 0)),
        ],
        out_specs=[
            pl.BlockSpec((1, _K, _RB), lambda p, rb: (p, 0, rb)),
            pl.BlockSpec((1, 1, N), lambda p, rb: (p, 0, 0)),
            pl.BlockSpec((1, 1, _RB), lambda p, rb: (p, 0, rb)),
            pl.BlockSpec((1, 3, N), lambda p, rb: (p, 0, 0)),
        ],
        out_shape=[
            jax.ShapeDtypeStruct((P, _K, N), jnp.int32),
            jax.ShapeDtypeStruct((P, 1, N), jnp.float32),
            jax.ShapeDtypeStruct((P, 1, N), jnp.float32),
            jax.ShapeDtypeStruct((P, 3, N), jnp.float32),
        ],
    )(pts, ptsT)


def _sort16(v):
    return plsc.sort_key_val(v, v)[0]


def _merge16(a, b):
    return jnp.minimum(a, jnp.flip(b))


def _sc_body(tab_hbm, ids_hbm, out_hbm, rx_v, ry_v, rz_v, pn_v, nq_v,
             ids_v, acc_v):
    wid = lax.axis_index("s") * 2 + lax.axis_index("c")
    wpp = _NW // (tab_hbm.shape[0] // 5)
    p5 = wid // wpp * 5
    pltpu.sync_copy(tab_hbm.at[p5], rx_v)
    pltpu.sync_copy(tab_hbm.at[p5 + 1], ry_v)
    pltpu.sync_copy(tab_hbm.at[p5 + 2], rz_v)
    pltpu.sync_copy(tab_hbm.at[p5 + 3], pn_v)
    pltpu.sync_copy(tab_hbm.at[p5 + 4], nq_v)
    pltpu.sync_copy(ids_hbm.at[wid], ids_v)
    lr0 = wid % wpp * (ids_v.shape[0] // _K)
    nrows = ids_v.shape[0] // _K
    zeros16 = jnp.zeros((16,), jnp.int32)

    def row_body(i, acc):
        ids16 = ids_v[pl.ds(i * _K, _K)]
        base = ids16 * _CH
        lrv = zeros16 + (lr0 + i)
        qnv = plsc.load_gather(nq_v, [lrv])
        cxv = plsc.load_gather(rx_v, [lrv]) * -2.0
        cyv = plsc.load_gather(ry_v, [lrv]) * -2.0
        czv = plsc.load_gather(rz_v, [lrv]) * -2.0
        vs = []
        for m in range(_CH):
            idx = base + m
            xm = plsc.load_gather(rx_v, [idx])
            ym = plsc.load_gather(ry_v, [idx])
            zm = plsc.load_gather(rz_v, [idx])
            nm = plsc.load_gather(pn_v, [idx])
            dot = (cxv * xm + cyv * ym) + czv * zm
            vs.append(_sort16((qnv + nm) + dot))
        while len(vs) > 2:
            vs = [_sort16(_merge16(vs[j], vs[j + 1]))
                  for j in range(0, len(vs), 2)]
        s = _merge16(vs[0], vs[1])
        return acc + jnp.maximum(s, 0.0)

    acc = lax.fori_loop(0, nrows, row_body, jnp.zeros((16,), jnp.float32))
    acc_v[...] = acc
    pltpu.sync_copy(acc_v, out_hbm.at[wid])


@jax.jit
def _sc_select(tab, ids3):
    nw, nrk = ids3.shape
    nrows = nrk // _K
    n = tab.shape[1]
    mesh = plsc.VectorSubcoreMesh(core_axis_name="c", subcore_axis_name="s")
    f = functools.partial(
        pl.kernel,
        mesh=mesh,
        out_type=jax.ShapeDtypeStruct((nw, 16), jnp.float32),
        compiler_params=pltpu.CompilerParams(
            needs_layout_passes=False, use_tc_tiling_on_sc=False),
        scratch_types=[
            pltpu.VMEM((n,), jnp.float32),
            pltpu.VMEM((n,), jnp.float32),
            pltpu.VMEM((n,), jnp.float32),
            pltpu.VMEM((n,), jnp.float32),
            pltpu.VMEM((n,), jnp.float32),
            pltpu.VMEM((nrows * _K,), jnp.int32),
            pltpu.VMEM((16,), jnp.float32),
        ],
    )(_sc_body)
    return f(tab, ids3)


def kernel(seed, gt_s):
    B, N, _ = seed.shape
    pts = jnp.concatenate([seed, gt_s], axis=0)
    ptsT = jnp.transpose(pts, (0, 2, 1))
    P = 2 * B
    sums = []
    for p in range(P):
        ids, pn, nqs, rpts = _tc_knn(pts[p:p + 1], ptsT[p:p + 1])
        tab = jnp.concatenate([rpts, pn, nqs], axis=1).reshape(5, N)
        ids3 = jnp.transpose(ids, (0, 2, 1)).reshape(_NW, N // _NW * _K)
        out = _sc_select(tab, ids3)
        sums.append(out.sum())
    per = jnp.stack(sums) / jnp.float32(N * _K)
    dis = per[:B]
    gt = per[B:]
    return jnp.mean((dis - gt) ** 2)


# speedup vs baseline: 1.0364x; 1.0364x over previous
import functools

import jax
import jax.numpy as jnp
from jax import lax
from jax.experimental import pallas as pl
from jax.experimental.pallas import tpu as pltpu
from jax.experimental.pallas import tpu_sc as plsc

_K = 16
_RB = 256
_CH = 16
_BIG = 3.0e38
_NW = 32


def _tc_body(q_ref, pT_ref, ids_ref, pn_ref, nq_ref, br_ref):
    q = q_ref[0]
    pT = pT_ref[0]
    n = pT.shape[1]
    nch = n // _CH
    qx = q[:, 0:1]
    qy = q[:, 1:2]
    qz = q[:, 2:3]
    px = pT[0:1, :]
    py = pT[1:2, :]
    pz = pT[2:3, :]
    nq = (qx * qx + qy * qy) + qz * qz
    np_ = (px * px + py * py) + pz * pz
    pn_ref[0] = np_
    nq_ref[0] = jnp.transpose(nq)
    bq = q.astype(jnp.bfloat16)
    bp = pT.astype(jnp.bfloat16)
    dot = lax.dot_general(bq, bp, (((1,), (0,)), ((), ())),
                          preferred_element_type=jnp.float32)
    x = (nq + np_) - 2.0 * dot
    br_ref[0] = bp.astype(jnp.float32)
    xT = jnp.transpose(x)
    x3 = xT.reshape(nch, _CH, _RB)
    cmT = jnp.min(x3, axis=1)
    iota = lax.broadcasted_iota(jnp.int32, (nch, _RB), 0)
    cols = []
    for _ in range(_K):
        mv = jnp.min(cmT, axis=0, keepdims=True)
        eq = cmT == mv
        idx = jnp.min(jnp.where(eq, iota, jnp.int32(1 << 30)), axis=0,
                      keepdims=True)
        cols.append(idx)
        cmT = jnp.where(iota == idx, _BIG, cmT)
    ids_ref[0] = jnp.concatenate(cols, axis=0)


@jax.jit
def _tc_knn(pts, ptsT):
    P, N, _ = pts.shape
    grid = (P, N // _RB)
    return pl.pallas_call(
        _tc_body,
        grid=grid,
        in_specs=[
            pl.BlockSpec((1, _RB, 3), lambda p, rb: (p, rb, 0)),
            pl.BlockSpec((1, 3, N), lambda p, rb: (p, 0, 0)),
        ],
        out_specs=[
            pl.BlockSpec((1, _K, _RB), lambda p, rb: (p, 0, rb)),
            pl.BlockSpec((1, 1, N), lambda p, rb: (p, 0, 0)),
            pl.BlockSpec((1, 1, _RB), lambda p, rb: (p, 0, rb)),
            pl.BlockSpec((1, 3, N), lambda p, rb: (p, 0, 0)),
        ],
        out_shape=[
            jax.ShapeDtypeStruct((P, _K, N), jnp.int32),
            jax.ShapeDtypeStruct((P, 1, N), jnp.float32),
            jax.ShapeDtypeStruct((P, 1, N), jnp.float32),
            jax.ShapeDtypeStruct((P, 3, N), jnp.float32),
        ],
    )(pts, ptsT)


def _sort16(v):
    return plsc.sort_key_val(v, v)[0]


def _merge16(a, b):
    return jnp.minimum(a, jnp.flip(b))


def _sc_body(tab_hbm, ids_hbm, out_hbm, rx_v, ry_v, rz_v, pn_v, nq_v,
             ids_v, acc_v):
    wid = lax.axis_index("s") * 2 + lax.axis_index("c")
    wpp = _NW // (tab_hbm.shape[0] // 5)
    p5 = wid // wpp * 5
    pltpu.sync_copy(tab_hbm.at[p5], rx_v)
    pltpu.sync_copy(tab_hbm.at[p5 + 1], ry_v)
    pltpu.sync_copy(tab_hbm.at[p5 + 2], rz_v)
    pltpu.sync_copy(tab_hbm.at[p5 + 3], pn_v)
    pltpu.sync_copy(tab_hbm.at[p5 + 4], nq_v)
    pltpu.sync_copy(ids_hbm.at[wid], ids_v)
    lr0 = wid % wpp * (ids_v.shape[0] // _K)
    nrows = ids_v.shape[0] // _K
    zeros16 = jnp.zeros((16,), jnp.int32)

    def row_body(i, acc):
        ids16 = ids_v[pl.ds(i * _K, _K)]
        base = ids16 * _CH
        lrv = zeros16 + (lr0 + i)
        qnv = plsc.load_gather(nq_v, [lrv])
        cxv = plsc.load_gather(rx_v, [lrv]) * -2.0
        cyv = plsc.load_gather(ry_v, [lrv]) * -2.0
        czv = plsc.load_gather(rz_v, [lrv]) * -2.0
        vs = []
        for m in range(_CH):
            idx = base + m
            xm = plsc.load_gather(rx_v, [idx])
            ym = plsc.load_gather(ry_v, [idx])
            zm = plsc.load_gather(rz_v, [idx])
            nm = plsc.load_gather(pn_v, [idx])
            dot = (cxv * xm + cyv * ym) + czv * zm
            vs.append(_sort16((qnv + nm) + dot))
        while len(vs) > 2:
            vs = [_sort16(_merge16(vs[j], vs[j + 1]))
                  for j in range(0, len(vs), 2)]
        s = _merge16(vs[0], vs[1])
        return acc + jnp.maximum(s, 0.0)

    acc = lax.fori_loop(0, nrows, row_body, jnp.zeros((16,), jnp.float32))
    acc_v[...] = acc
    pltpu.sync_copy(acc_v, out_hbm.at[wid])


@jax.jit
def _sc_select(tab, ids3):
    nw, nrk = ids3.shape
    nrows = nrk // _K
    n = tab.shape[1]
    mesh = plsc.VectorSubcoreMesh(core_axis_name="c", subcore_axis_name="s")
    f = functools.partial(
        pl.kernel,
        mesh=mesh,
        out_type=jax.ShapeDtypeStruct((nw, 16), jnp.float32),
        compiler_params=pltpu.CompilerParams(
            needs_layout_passes=False, use_tc_tiling_on_sc=False),
        scratch_types=[
            pltpu.VMEM((n,), jnp.float32),
            pltpu.VMEM((n,), jnp.float32),
            pltpu.VMEM((n,), jnp.float32),
            pltpu.VMEM((n,), jnp.float32),
            pltpu.VMEM((n,), jnp.float32),
            pltpu.VMEM((nrows * _K,), jnp.int32),
            pltpu.VMEM((16,), jnp.float32),
        ],
    )(_sc_body)
    return f(tab, ids3)


def kernel(seed, gt_s):
    B, N, _ = seed.shape
    pts = jnp.concatenate([seed, gt_s], axis=0)
    ptsT = jnp.transpose(pts, (0, 2, 1))
    P = 2 * B
    sums = []
    for p in range(P):
        ids, pn, nqs, rpts = _tc_knn(pts[p:p + 1], ptsT[p:p + 1])
        tab = jnp.concatenate([rpts, pn, nqs], axis=1).reshape(5, N)
        ids3 = jnp.transpose(ids, (0, 2, 1)).reshape(_NW, N // _NW * _K)
        out = _sc_select(tab, ids3)
        sums.append(out.sum())
    per = jnp.stack(sums) / jnp.float32(N * _K)
    dis = per[:B]
    gt = per[B:]
    return jnp.mean((dis - gt) ** 2)
